# Initial kernel scaffold; baseline (speedup 1.0000x reference)
import jax
import jax.numpy as jnp
from jax.experimental import pallas as pl


def kernel(x, edge_index, enc_W1, enc_b1, enc_W2, enc_b2, g1_W, g1_b, g2_W, g2_b, g3_W, g3_b):
    raise NotImplementedError("write your pallas kernel here")


# speedup vs baseline: 8.9828x; 8.9828x over previous
import functools

import jax
import jax.numpy as jnp
from jax import lax
from jax.experimental import pallas as pl
from jax.experimental.pallas import tpu as pltpu
from jax.experimental.pallas import tpu_sc as plsc

N = 10000
E = 320000
IN1 = 128
IN2 = 144
HID = 128

NPAD = 10240
NC, NS = 2, 16
NWORK = NC * NS
CHUNK = 128
EPT_CHUNKS = 79
TOTAL_CHUNKS = NWORK * EPT_CHUNKS
E_PAD = TOTAL_CHUNKS * CHUNK
ROWS_PER_TILE = NPAD // NS
BLK = 1024
DEGW = 128

@functools.lru_cache(maxsize=None)
def _mesh():
    return plsc.VectorSubcoreMesh(
        core_axis_name="c", subcore_axis_name="s",
        num_cores=NC, num_subcores=NS)


def _sc_degree(edges, zeros8, ones8):
    @functools.partial(
        pl.kernel,
        out_type=jax.ShapeDtypeStruct((NC, NPAD, DEGW), jnp.float32),
        mesh=_mesh(),
        scratch_types=[
            pltpu.VMEM_SHARED((NPAD, DEGW), jnp.float32),
            pltpu.VMEM((2, CHUNK), jnp.int32),
            pltpu.VMEM((CHUNK, DEGW), jnp.float32),
        ],
    )
    def deg_kernel(edges_hbm, zeros_hbm, ones_hbm, out_hbm, deg_sp, idx_v, ones_v):
        c = lax.axis_index("c")
        s = lax.axis_index("s")
        t = c * NS + s
        r0 = s * ROWS_PER_TILE
        pltpu.sync_copy(zeros_hbm.at[pl.ds(r0, ROWS_PER_TILE)],
                        deg_sp.at[pl.ds(r0, ROWS_PER_TILE)])
        pltpu.sync_copy(ones_hbm, ones_v)
        plsc.subcore_barrier()

        def body(k, carry):
            pltpu.sync_copy(edges_hbm.at[t * EPT_CHUNKS + k], idx_v)
            pltpu.sync_copy(ones_v, deg_sp.at[idx_v.at[1]], add=True)
            return carry

        lax.fori_loop(0, EPT_CHUNKS, body, 0)
        plsc.subcore_barrier()
        pltpu.sync_copy(deg_sp.at[pl.ds(r0, ROWS_PER_TILE)],
                        out_hbm.at[c, pl.ds(r0, ROWS_PER_TILE)])

    return deg_kernel(edges, zeros8, ones8)


def _sc_conv(hws, edges, zeros_hid):
    @functools.partial(
        pl.kernel,
        out_type=jax.ShapeDtypeStruct((NC, NPAD, HID), jnp.float32),
        mesh=_mesh(),
        scratch_types=[
            pltpu.VMEM_SHARED((NPAD, HID), jnp.float32),
            pltpu.VMEM((2, CHUNK), jnp.int32),
            pltpu.VMEM((CHUNK, HID), jnp.float32),
            pltpu.SemaphoreType.DMA,
        ],
    )
    def conv_kernel(hws_hbm, edges_hbm, zeros_hbm, out_hbm,
                    acc_sp, idx_v, rows_v, sem):
        c = lax.axis_index("c")
        s = lax.axis_index("s")
        t = c * NS + s
        r0 = s * ROWS_PER_TILE
        pltpu.sync_copy(zeros_hbm.at[pl.ds(r0, ROWS_PER_TILE)],
                        acc_sp.at[pl.ds(r0, ROWS_PER_TILE)])
        plsc.subcore_barrier()

        def body(k, carry):
            pltpu.sync_copy(edges_hbm.at[t * EPT_CHUNKS + k], idx_v)
            pltpu.async_copy(hws_hbm.at[idx_v.at[0]], rows_v, sem).wait()
            pltpu.sync_copy(rows_v, acc_sp.at[idx_v.at[1]], add=True)
            return carry

        lax.fori_loop(0, EPT_CHUNKS, body, 0)
        plsc.subcore_barrier()
        pltpu.sync_copy(acc_sp.at[pl.ds(r0, ROWS_PER_TILE)],
                        out_hbm.at[c, pl.ds(r0, ROWS_PER_TILE)])

    return conv_kernel(hws, edges, zeros_hid)


def _tc_stage1(x_p, deg0, deg1, w1, b1, w2, b2, g1w):
    def body(x_ref, d0_ref, d1_ref, w1_ref, b1_ref, w2_ref, b2_ref, g_ref,
             hws_ref, dinv_ref):
        xb = x_ref[...]
        d = 1.0 + d0_ref[...][:, :1] + d1_ref[...][:, :1]
        dinv = lax.rsqrt(d)
        t = jnp.maximum(
            jnp.dot(xb[:, :IN1], w1_ref[...], preferred_element_type=jnp.float32)
            + b1_ref[...], 0.0)
        t = jnp.maximum(
            jnp.dot(t, w2_ref[...], preferred_element_type=jnp.float32)
            + b2_ref[...], 0.0)
        g = g_ref[...]
        hw = (jnp.dot(t, g[:IN1], preferred_element_type=jnp.float32)
              + jnp.dot(xb[:, IN1:], g[IN1:], preferred_element_type=jnp.float32))
        hws_ref[...] = hw * dinv
        dinv_ref[...] = jnp.broadcast_to(dinv, (BLK, 8))

    grid = (NPAD // BLK,)
    return pl.pallas_call(
        body,
        grid=grid,
        in_specs=[
            pl.BlockSpec((BLK, IN2), lambda i: (i, 0)),
            pl.BlockSpec((BLK, DEGW), lambda i: (i, 0)),
            pl.BlockSpec((BLK, DEGW), lambda i: (i, 0)),
            pl.BlockSpec((IN1, HID), lambda i: (0, 0)),
            pl.BlockSpec((1, HID), lambda i: (0, 0)),
            pl.BlockSpec((HID, HID), lambda i: (0, 0)),
            pl.BlockSpec((1, HID), lambda i: (0, 0)),
            pl.BlockSpec((IN2, HID), lambda i: (0, 0)),
        ],
        out_specs=[
            pl.BlockSpec((BLK, HID), lambda i: (i, 0)),
            pl.BlockSpec((BLK, 8), lambda i: (i, 0)),
        ],
        out_shape=[
            jax.ShapeDtypeStruct((NPAD, HID), jnp.float32),
            jax.ShapeDtypeStruct((NPAD, 8), jnp.float32),
        ],
    )(x_p, deg0, deg1, w1, b1, w2, b2, g1w)


def _tc_mid(a0, a1, hws, dinv8, w, b):
    def body(a0_ref, a1_ref, hws_ref, dinv_ref, w_ref, b_ref, out_ref):
        dinv = dinv_ref[...][:, :1]
        h = jnp.maximum(
            dinv * (a0_ref[...] + a1_ref[...] + hws_ref[...]) + b_ref[...], 0.0)
        out_ref[...] = jnp.dot(
            h, w_ref[...], preferred_element_type=jnp.float32) * dinv

    return pl.pallas_call(
        body,
        grid=(NPAD // BLK,),
        in_specs=[
            pl.BlockSpec((BLK, HID), lambda i: (i, 0)),
            pl.BlockSpec((BLK, HID), lambda i: (i, 0)),
            pl.BlockSpec((BLK, HID), lambda i: (i, 0)),
            pl.BlockSpec((BLK, 8), lambda i: (i, 0)),
            pl.BlockSpec((HID, HID), lambda i: (0, 0)),
            pl.BlockSpec((1, HID), lambda i: (0, 0)),
        ],
        out_specs=pl.BlockSpec((BLK, HID), lambda i: (i, 0)),
        out_shape=jax.ShapeDtypeStruct((NPAD, HID), jnp.float32),
    )(a0, a1, hws, dinv8, w, b)


def _tc_final(a0, a1, hws, dinv8, b):
    def body(a0_ref, a1_ref, hws_ref, dinv_ref, b_ref, out_ref):
        dinv = dinv_ref[...][:, :1]
        out_ref[...] = (
            dinv * (a0_ref[...] + a1_ref[...] + hws_ref[...]) + b_ref[...])

    return pl.pallas_call(
        body,
        grid=(NPAD // BLK,),
        in_specs=[
            pl.BlockSpec((BLK, HID), lambda i: (i, 0)),
            pl.BlockSpec((BLK, HID), lambda i: (i, 0)),
            pl.BlockSpec((BLK, HID), lambda i: (i, 0)),
            pl.BlockSpec((BLK, 8), lambda i: (i, 0)),
            pl.BlockSpec((1, HID), lambda i: (0, 0)),
        ],
        out_specs=pl.BlockSpec((BLK, HID), lambda i: (i, 0)),
        out_shape=jax.ShapeDtypeStruct((NPAD, HID), jnp.float32),
    )(a0, a1, hws, dinv8, b)


def kernel(x, edge_index, enc_W1, enc_b1, enc_W2, enc_b2,
           g1_W, g1_b, g2_W, g2_b, g3_W, g3_b):
    pad = E_PAD - E
    src_p = jnp.concatenate(
        [edge_index[0].astype(jnp.int32), jnp.zeros((pad,), jnp.int32)])
    dst_p = jnp.concatenate(
        [edge_index[1].astype(jnp.int32), jnp.full((pad,), N, jnp.int32)])
    edges = (jnp.stack([src_p, dst_p])
             .reshape(2, TOTAL_CHUNKS, CHUNK)
             .transpose(1, 0, 2))
    x_p = jnp.pad(x, ((0, NPAD - N), (0, 0)))
    zeros8 = jnp.zeros((NPAD, DEGW), jnp.float32)
    ones8 = jnp.ones((CHUNK, DEGW), jnp.float32)
    zeros_hid = jnp.zeros((NPAD, HID), jnp.float32)

    degp = _sc_degree(edges, zeros8, ones8)
    hws1, dinv8 = _tc_stage1(
        x_p, degp[0], degp[1], enc_W1, enc_b1.reshape(1, -1),
        enc_W2, enc_b2.reshape(1, -1), g1_W)
    acc1 = _sc_conv(hws1, edges, zeros_hid)
    hws2 = _tc_mid(acc1[0], acc1[1], hws1, dinv8, g2_W, g1_b.reshape(1, -1))
    acc2 = _sc_conv(hws2, edges, zeros_hid)
    hws3 = _tc_mid(acc2[0], acc2[1], hws2, dinv8, g3_W, g2_b.reshape(1, -1))
    acc3 = _sc_conv(hws3, edges, zeros_hid)
    out = _tc_final(acc3[0], acc3[1], hws3, dinv8, g3_b.reshape(1, -1))
    return out[:N]
